# baseline (speedup 1.0000x reference)
import jax
import jax.numpy as jnp
from jax.experimental import pallas as pl
from jax.experimental.pallas import tpu as pltpu

_CHUNK = 512
_NB = 8


def _emb_block(bounds_ref, emb_ref, pos0, rows):
    pos = jax.lax.broadcasted_iota(jnp.int32, (rows, 1), 0) + pos0
    t0 = emb_ref[0:1, :]
    t1 = emb_ref[1:2, :]
    t2 = emb_ref[2:3, :]
    return jnp.where(
        pos < bounds_ref[0],
        t0,
        jnp.where(pos < bounds_ref[1], t1, jnp.where(pos < bounds_ref[2], t0, t2)),
    )


def _make_body(R, n, d):
    nch = R // _CHUNK

    def body(bounds_ref, x_hbm, emb_ref, out_hbm, ibufs, obufs, isems, osems):
        def start_in(c):
            b = c % _NB
            pltpu.make_async_copy(
                x_hbm.at[pl.ds(c * _CHUNK, _CHUNK), :], ibufs.at[b], isems.at[b]
            ).start()

        def wait_in(c):
            b = c % _NB
            pltpu.make_async_copy(
                x_hbm.at[pl.ds(c * _CHUNK, _CHUNK), :], ibufs.at[b], isems.at[b]
            ).wait()

        def start_out(c):
            b = c % _NB
            pltpu.make_async_copy(
                obufs.at[b], out_hbm.at[pl.ds(c * _CHUNK, _CHUNK), :], osems.at[b]
            ).start()

        def wait_out(c):
            b = c % _NB
            pltpu.make_async_copy(
                obufs.at[b], out_hbm.at[pl.ds(c * _CHUNK, _CHUNK), :], osems.at[b]
            ).wait()

        for c in range(min(_NB, nch)):
            start_in(c)
        for c in range(nch):
            b = c % _NB
            wait_in(c)
            if c >= _NB:
                wait_out(c - _NB)
            pos0 = (c * _CHUNK) % n
            obufs[b] = ibufs[b] + _emb_block(bounds_ref, emb_ref, pos0, _CHUNK)
            start_out(c)
            if c + _NB < nch:
                start_in(c + _NB)
        for c in range(max(nch - _NB, 0), nch):
            wait_out(c)

    return body


def kernel(x, emb_table, seq_len, front, back, keyframe_gap):
    batch, n, d = x.shape
    R = batch * n
    seq_len = jnp.asarray(seq_len, jnp.int32)
    front = jnp.asarray(front, jnp.int32)
    back = jnp.asarray(back, jnp.int32)
    keyframe_gap = jnp.asarray(keyframe_gap, jnp.int32)
    ignored_len = seq_len - front - back - keyframe_gap
    bounds = jnp.stack(
        [front, front + keyframe_gap, seq_len - ignored_len], axis=0
    ).astype(jnp.int32)

    out = pl.pallas_call(
        _make_body(R, n, d),
        in_specs=[
            pl.BlockSpec(memory_space=pltpu.SMEM),
            pl.BlockSpec(memory_space=pl.ANY),
            pl.BlockSpec(memory_space=pltpu.VMEM),
        ],
        out_specs=pl.BlockSpec(memory_space=pl.ANY),
        out_shape=jax.ShapeDtypeStruct((R, d), x.dtype),
        scratch_shapes=[
            pltpu.VMEM((_NB, _CHUNK, d), jnp.float32),
            pltpu.VMEM((_NB, _CHUNK, d), jnp.float32),
            pltpu.SemaphoreType.DMA((_NB,)),
            pltpu.SemaphoreType.DMA((_NB,)),
        ],
    )(bounds, x.reshape(R, d), emb_table)
    return out.reshape(batch, n, d)


# speedup vs baseline: 1.0081x; 1.0081x over previous
import jax
import jax.numpy as jnp
from jax.experimental import pallas as pl
from jax.experimental.pallas import tpu as pltpu

_CHUNK = 1024
_NB = 4


def _emb_block(bounds_ref, emb_ref, pos0, rows):
    pos = jax.lax.broadcasted_iota(jnp.int32, (rows, 1), 0) + pos0
    t0 = emb_ref[0:1, :]
    t1 = emb_ref[1:2, :]
    t2 = emb_ref[2:3, :]
    return jnp.where(
        pos < bounds_ref[0],
        t0,
        jnp.where(pos < bounds_ref[1], t1, jnp.where(pos < bounds_ref[2], t0, t2)),
    )


def _make_body(R, n, d):
    nch = R // _CHUNK

    def body(bounds_ref, x_hbm, emb_ref, out_hbm, ibufs, obufs, isems, osems):
        def start_in(c):
            b = c % _NB
            pltpu.make_async_copy(
                x_hbm.at[pl.ds(c * _CHUNK, _CHUNK), :], ibufs.at[b], isems.at[b]
            ).start()

        def wait_in(c):
            b = c % _NB
            pltpu.make_async_copy(
                x_hbm.at[pl.ds(c * _CHUNK, _CHUNK), :], ibufs.at[b], isems.at[b]
            ).wait()

        def start_out(c):
            b = c % _NB
            pltpu.make_async_copy(
                obufs.at[b], out_hbm.at[pl.ds(c * _CHUNK, _CHUNK), :], osems.at[b]
            ).start()

        def wait_out(c):
            b = c % _NB
            pltpu.make_async_copy(
                obufs.at[b], out_hbm.at[pl.ds(c * _CHUNK, _CHUNK), :], osems.at[b]
            ).wait()

        for c in range(min(_NB, nch)):
            start_in(c)
        for c in range(nch):
            b = c % _NB
            wait_in(c)
            if c >= _NB:
                wait_out(c - _NB)
            pos0 = (c * _CHUNK) % n
            obufs[b] = ibufs[b] + _emb_block(bounds_ref, emb_ref, pos0, _CHUNK)
            start_out(c)
            if c + _NB < nch:
                start_in(c + _NB)
        for c in range(max(nch - _NB, 0), nch):
            wait_out(c)

    return body


def kernel(x, emb_table, seq_len, front, back, keyframe_gap):
    batch, n, d = x.shape
    R = batch * n
    seq_len = jnp.asarray(seq_len, jnp.int32)
    front = jnp.asarray(front, jnp.int32)
    back = jnp.asarray(back, jnp.int32)
    keyframe_gap = jnp.asarray(keyframe_gap, jnp.int32)
    ignored_len = seq_len - front - back - keyframe_gap
    bounds = jnp.stack(
        [front, front + keyframe_gap, seq_len - ignored_len], axis=0
    ).astype(jnp.int32)

    out = pl.pallas_call(
        _make_body(R, n, d),
        in_specs=[
            pl.BlockSpec(memory_space=pltpu.SMEM),
            pl.BlockSpec(memory_space=pl.ANY),
            pl.BlockSpec(memory_space=pltpu.VMEM),
        ],
        out_specs=pl.BlockSpec(memory_space=pl.ANY),
        out_shape=jax.ShapeDtypeStruct((R, d), x.dtype),
        scratch_shapes=[
            pltpu.VMEM((_NB, _CHUNK, d), jnp.float32),
            pltpu.VMEM((_NB, _CHUNK, d), jnp.float32),
            pltpu.SemaphoreType.DMA((_NB,)),
            pltpu.SemaphoreType.DMA((_NB,)),
        ],
    )(bounds, x.reshape(R, d), emb_table)
    return out.reshape(batch, n, d)
